# Initial kernel scaffold; baseline (speedup 1.0000x reference)
import jax
import jax.numpy as jnp
from jax.experimental import pallas as pl


def kernel(x, edge_index, W0, as0, ad0, b0, g0, be0, W1, as1, ad1, b1, g1, be1, Wc, bc):
    raise NotImplementedError("write your pallas kernel here")


# speedup vs baseline: 38.0990x; 38.0990x over previous
import functools

import jax
import jax.numpy as jnp
from jax import lax
from jax.experimental import pallas as pl
from jax.experimental.pallas import tpu as pltpu
from jax.experimental.pallas import tpu_sc as plsc

N = 10000
E = 320000
IN = 128
HID = 16
HEADS = 8
F = HEADS * HID
NEG = 0.2

NC = 2
NS = 16
L = 16
NW = NC * NS

EPW = 10368
ET = EPW * NW
NP = 10112
RPT = NP // NS
C1 = 128
C2 = 64
NCH1 = EPW // C1
NCH2 = EPW // C2

f32 = jnp.float32
i32 = jnp.int32


_RB = 400


def _embed_body(x_ref, w_ref, a_ref, h_ref, asd_ref, cm_ref, mx_ref):
    i = pl.program_id(0)
    h = jnp.dot(x_ref[...], w_ref[...], preferred_element_type=f32)
    h_ref[...] = h
    asd = jnp.dot(h, a_ref[...], preferred_element_type=f32)
    asd_ref[...] = asd
    bm = jnp.max(asd, axis=0, keepdims=True)

    @pl.when(i == 0)
    def _():
        mx_ref[...] = bm

    @pl.when(i > 0)
    def _():
        mx_ref[...] = jnp.maximum(mx_ref[...], bm)

    @pl.when(i == pl.num_programs(0) - 1)
    def _():
        m = mx_ref[...]
        c = m[:, :HEADS] + m[:, HEADS:]
        cl = jnp.maximum(c, NEG * c)
        cm_ref[...] = jnp.concatenate([cl, jnp.zeros_like(cl)], axis=1)


def _tc_embed(x, w, aall):
    return pl.pallas_call(
        _embed_body,
        grid=(N // _RB,),
        in_specs=[
            pl.BlockSpec((_RB, IN), lambda i: (i, 0)),
            pl.BlockSpec((IN, F), lambda i: (0, 0)),
            pl.BlockSpec((F, 2 * HEADS), lambda i: (0, 0)),
        ],
        out_specs=[
            pl.BlockSpec((_RB, F), lambda i: (i, 0)),
            pl.BlockSpec((_RB, 2 * HEADS), lambda i: (i, 0)),
            pl.BlockSpec((1, 2 * HEADS), lambda i: (0, 0)),
        ],
        out_shape=[
            jax.ShapeDtypeStruct((N, F), f32),
            jax.ShapeDtypeStruct((N, 2 * HEADS), f32),
            jax.ShapeDtypeStruct((1, 2 * HEADS), f32),
        ],
        scratch_shapes=[pltpu.VMEM((1, 2 * HEADS), f32)],
    )(x, w, aall)


def _stat_body(op_ref, dp_ref, r8_ref, b_ref, s1_ref, s2_ref):
    inv = 1.0 / (dp_ref[0, :N, :] + dp_ref[1, :N, :] + 1e-16)
    scale = jnp.dot(inv, r8_ref[...], preferred_element_type=f32)
    o = (op_ref[0, :N, :] + op_ref[1, :N, :]) * scale + b_ref[...]
    s1_ref[...] = jnp.sum(o, axis=0, keepdims=True)
    s2_ref[...] = jnp.sum(o * o, axis=0, keepdims=True)


def _tc_stat(op, dp, r8, b):
    d = b.shape[1]
    return pl.pallas_call(
        _stat_body,
        out_shape=[
            jax.ShapeDtypeStruct((1, d), f32),
            jax.ShapeDtypeStruct((1, d), f32),
        ],
    )(op, dp, r8, b)


def _main1_body(op_ref, dp_ref, r8_ref, b_ref, g_ref, be_ref, s1_ref, s2_ref,
                w_ref, a_ref, h_ref, asd_ref, cm_ref, mx_ref):
    i = pl.program_id(0)
    inv = 1.0 / (dp_ref[0] + dp_ref[1] + 1e-16)
    scale = jnp.dot(inv, r8_ref[...], preferred_element_type=f32)
    o = (op_ref[0] + op_ref[1]) * scale + b_ref[...]
    m = s1_ref[...] / N
    v = s2_ref[...] / N - m * m
    xn = g_ref[...] * (o - m) * lax.rsqrt(v + 1e-5) + be_ref[...]
    el = jnp.where(xn > 0, xn, jnp.exp(xn) - 1.0)
    h = jnp.dot(el, w_ref[...], preferred_element_type=f32)
    h_ref[...] = h
    asd = jnp.dot(h, a_ref[...], preferred_element_type=f32)
    asd_ref[...] = asd
    bm = jnp.max(asd, axis=0, keepdims=True)

    @pl.when(i == 0)
    def _():
        mx_ref[...] = bm

    @pl.when(i > 0)
    def _():
        mx_ref[...] = jnp.maximum(mx_ref[...], bm)

    @pl.when(i == pl.num_programs(0) - 1)
    def _():
        mm = mx_ref[...]
        c = mm[:, :HEADS] + mm[:, HEADS:]
        cl = jnp.maximum(c, NEG * c)
        cm_ref[...] = jnp.concatenate([cl, jnp.zeros_like(cl)], axis=1)


def _tc_main1(op, dp, r8, b, g, be, s1, s2, w, aall):
    return pl.pallas_call(
        _main1_body,
        grid=(N // _RB,),
        in_specs=[
            pl.BlockSpec((2, _RB, F), lambda i: (0, i, 0)),
            pl.BlockSpec((2, _RB, HEADS), lambda i: (0, i, 0)),
            pl.BlockSpec((HEADS, F), lambda i: (0, 0)),
            pl.BlockSpec((1, F), lambda i: (0, 0)),
            pl.BlockSpec((1, F), lambda i: (0, 0)),
            pl.BlockSpec((1, F), lambda i: (0, 0)),
            pl.BlockSpec((1, F), lambda i: (0, 0)),
            pl.BlockSpec((1, F), lambda i: (0, 0)),
            pl.BlockSpec((F, F), lambda i: (0, 0)),
            pl.BlockSpec((F, 2 * HEADS), lambda i: (0, 0)),
        ],
        out_specs=[
            pl.BlockSpec((_RB, F), lambda i: (i, 0)),
            pl.BlockSpec((_RB, 2 * HEADS), lambda i: (i, 0)),
            pl.BlockSpec((1, 2 * HEADS), lambda i: (0, 0)),
        ],
        out_shape=[
            jax.ShapeDtypeStruct((N, F), f32),
            jax.ShapeDtypeStruct((N, 2 * HEADS), f32),
            jax.ShapeDtypeStruct((1, 2 * HEADS), f32),
        ],
        scratch_shapes=[pltpu.VMEM((1, 2 * HEADS), f32)],
    )(op, dp, r8, b, g, be, s1, s2, w, aall)


def _statf_body(op_ref, dp_ref, r8_ref, mavg_ref, b_ref, s1_ref, s2_ref):
    inv = 1.0 / (dp_ref[0, :N, :] + dp_ref[1, :N, :] + 1e-16)
    scale = jnp.dot(inv, r8_ref[...], preferred_element_type=f32)
    o = (op_ref[0, :N, :] + op_ref[1, :N, :]) * scale
    hm = jnp.dot(o, mavg_ref[...], preferred_element_type=f32) + b_ref[...]
    s1_ref[...] = jnp.sum(hm, axis=0, keepdims=True)
    s2_ref[...] = jnp.sum(hm * hm, axis=0, keepdims=True)


def _tc_statf(op, dp, r8, mavg, b):
    return pl.pallas_call(
        _statf_body,
        out_shape=[
            jax.ShapeDtypeStruct((1, HID), f32),
            jax.ShapeDtypeStruct((1, HID), f32),
        ],
    )(op, dp, r8, mavg, b)


def _final_body(op_ref, dp_ref, r8_ref, mavg_ref, b_ref, g_ref, be_ref,
                s1_ref, s2_ref, wc_ref, bc_ref, out_ref):
    inv = 1.0 / (dp_ref[0, :N, :] + dp_ref[1, :N, :] + 1e-16)
    scale = jnp.dot(inv, r8_ref[...], preferred_element_type=f32)
    o = (op_ref[0, :N, :] + op_ref[1, :N, :]) * scale
    hm = jnp.dot(o, mavg_ref[...], preferred_element_type=f32) + b_ref[...]
    m = s1_ref[...] / N
    v = s2_ref[...] / N - m * m
    xn = g_ref[...] * (hm - m) * lax.rsqrt(v + 1e-5) + be_ref[...]
    out_ref[...] = jnp.dot(xn, wc_ref[...], preferred_element_type=f32) + bc_ref[...]


def _tc_final(op, dp, r8, mavg, b, g, be, s1, s2, wc, bc):
    return pl.pallas_call(
        _final_body,
        out_shape=jax.ShapeDtypeStruct((N, 2), f32),
    )(op, dp, r8, mavg, b, g, be, s1, s2, wc, bc)


_MESH = plsc.VectorSubcoreMesh(
    core_axis_name="c", subcore_axis_name="s", num_cores=NC, num_subcores=NS)


def _sc_pass1_body(src_h, dst_h, alo_h, ahi_h, dlo_h, dhi_h, cm_h,
                   exa_h, exb_h, dp_h,
                   atab, dtab, sidx, didx, exbuf, cm_v, den_sh):
    c = lax.axis_index("c")
    s = lax.axis_index("s")
    wid = c * NS + s
    ebase = wid * EPW
    iota = lax.iota(i32, L)
    zero = jnp.zeros((L,), f32)

    pltpu.sync_copy(cm_h, cm_v)
    cmvec = cm_v[...]

    for k in range(C1 * HEADS // L):
        flat = iota + k * L
        plsc.store_scatter(exbuf, [flat // HEADS, flat % HEADS], zero)
    for r in range(0, RPT, C1):
        rows = min(C1, RPT - r)
        pltpu.sync_copy(exbuf.at[pl.ds(0, rows), :],
                        den_sh.at[pl.ds(s * RPT + r, rows), :])
    plsc.subcore_barrier()

    for phase in range(2):
        if phase == 0:
            pltpu.sync_copy(alo_h, atab)
            pltpu.sync_copy(dlo_h, dtab)
        else:
            for k in range(C1 * HEADS // L):
                flat = iota + k * L
                plsc.store_scatter(exbuf, [flat // HEADS, flat % HEADS], zero)
            pltpu.sync_copy(ahi_h, atab)
            pltpu.sync_copy(dhi_h, dtab)
        col0 = phase * 4
        ex_out = exa_h if phase == 0 else exb_h

        def chunk_body(ci, carry):
            base = ebase + ci * C1
            pltpu.sync_copy(src_h.at[pl.ds(base, C1)], sidx)
            pltpu.sync_copy(dst_h.at[pl.ds(base, C1)], didx)
            for g in range(C1 // L):
                sv = sidx[pl.ds(g * L, L)]
                dv = didx[pl.ds(g * L, L)]
                ri = iota + g * L
                for hh in range(4):
                    av = plsc.load_gather(atab, [sv * 4 + hh])
                    bv = plsc.load_gather(dtab, [dv * 4 + hh])
                    e = av + bv
                    e = jnp.maximum(e, NEG * e)
                    ex = jnp.exp(e - cmvec[col0 + hh])
                    plsc.store_scatter(
                        exbuf, [ri, jnp.full((L,), col0 + hh, i32)], ex)
            pltpu.sync_copy(exbuf, ex_out.at[pl.ds(base, C1), :])
            pltpu.sync_copy(exbuf, den_sh.at[didx], add=True)
            return carry

        lax.fori_loop(0, NCH1, chunk_body, 0)

    plsc.subcore_barrier()
    pltpu.sync_copy(den_sh.at[pl.ds(s * RPT, RPT), :],
                    dp_h.at[c, pl.ds(s * RPT, RPT), :])


_sc_pass1 = functools.partial(
    pl.kernel,
    out_type=(
        jax.ShapeDtypeStruct((ET, HEADS), f32),
        jax.ShapeDtypeStruct((ET, HEADS), f32),
        jax.ShapeDtypeStruct((NC, NP, HEADS), f32),
    ),
    mesh=_MESH,
    compiler_params=pltpu.CompilerParams(
        needs_layout_passes=False, use_tc_tiling_on_sc=False),
    scratch_types=[
        pltpu.VMEM((N * 4,), f32),
        pltpu.VMEM((N * 4,), f32),
        pltpu.VMEM((C1,), i32),
        pltpu.VMEM((C1,), i32),
        pltpu.VMEM((C1, HEADS), f32),
        pltpu.VMEM((L,), f32),
        pltpu.VMEM_SHARED((NP, HEADS), f32),
    ],
)(_sc_pass1_body)


def _sc_pass2_body(src_h, dst_h, h_hbm, exa_h, exb_h,
                   op_h,
                   sidx, didx, exba, exbb, alphab, hrows, msgb,
                   sem, out_sh):
    c = lax.axis_index("c")
    s = lax.axis_index("s")
    wid = c * NS + s
    ebase = wid * EPW
    iota = lax.iota(i32, L)
    zero = jnp.zeros((L,), f32)

    for r in range(C2):
        for g in range(F // L):
            msgb[r, pl.ds(g * L, L)] = zero
    for r in range(0, RPT, C2):
        rows = min(C2, RPT - r)
        pltpu.sync_copy(msgb.at[pl.ds(0, rows), :],
                        out_sh.at[pl.ds(s * RPT + r, rows), :])
    plsc.subcore_barrier()

    def chunk_body(ci, carry):
        base = ebase + ci * C2
        pltpu.sync_copy(src_h.at[pl.ds(base, C2)], sidx)
        pltpu.sync_copy(dst_h.at[pl.ds(base, C2)], didx)
        pltpu.sync_copy(exa_h.at[pl.ds(base, C2), :], exba)
        pltpu.sync_copy(exb_h.at[pl.ds(base, C2), :], exbb)
        pltpu.async_copy(h_hbm.at[sidx], hrows, sem).wait()
        for g in range(C2 // L):
            rv = iota + g * L
            for h in range(HEADS):
                hsp = jnp.full((L,), h, i32)
                exv = (plsc.load_gather(exba, [rv, hsp]) +
                       plsc.load_gather(exbb, [rv, hsp]))
                plsc.store_scatter(alphab, [rv, hsp], exv)

        def edge_body(e, cc):
            arow = alphab[e, :]
            for g in range(HEADS):
                msgb[e, pl.ds(g * HID, HID)] = (
                    arow[g] * hrows[e, pl.ds(g * HID, HID)])
            return cc

        lax.fori_loop(0, C2, edge_body, 0)
        pltpu.sync_copy(msgb, out_sh.at[didx], add=True)
        return carry

    lax.fori_loop(0, NCH2, chunk_body, 0)

    plsc.subcore_barrier()
    pltpu.sync_copy(out_sh.at[pl.ds(s * RPT, RPT), :],
                    op_h.at[c, pl.ds(s * RPT, RPT), :])


_sc_pass2 = functools.partial(
    pl.kernel,
    out_type=jax.ShapeDtypeStruct((NC, NP, F), f32),
    mesh=_MESH,
    compiler_params=pltpu.CompilerParams(
        needs_layout_passes=False, use_tc_tiling_on_sc=False),
    scratch_types=[
        pltpu.VMEM((C2,), i32),
        pltpu.VMEM((C2,), i32),
        pltpu.VMEM((C2, HEADS), f32),
        pltpu.VMEM((C2, HEADS), f32),
        pltpu.VMEM((C2, L), f32),
        pltpu.VMEM((C2, F), f32),
        pltpu.VMEM((C2, F), f32),
        pltpu.SemaphoreType.DMA,
        pltpu.VMEM_SHARED((NP, F), f32),
    ],
)(_sc_pass2_body)


def _make_aall(a_src, a_dst):
    jj = jnp.arange(F)
    blk = (jj[:, None] // HID == jnp.arange(HEADS)[None, :]).astype(f32)
    return jnp.concatenate(
        [blk * a_src.reshape(F)[:, None], blk * a_dst.reshape(F)[:, None]],
        axis=1)


def _sc_layer(h, asd, cm, src, dst):
    alo = asd[:, 0:4].reshape(-1)
    ahi = asd[:, 4:8].reshape(-1)
    dlo = asd[:, 8:12].reshape(-1)
    dhi = asd[:, 12:16].reshape(-1)
    cmv = cm.reshape(2 * HEADS)
    exa, exb, dp = _sc_pass1(src, dst, alo, ahi, dlo, dhi, cmv)
    return _sc_pass2(src, dst, h, exa, exb), dp


def kernel(x, edge_index, W0, as0, ad0, b0, g0, be0,
           W1, as1, ad1, b1, g1, be1, Wc, bc):
    pad = ET - E - N
    loops = jnp.arange(N, dtype=i32)
    src = jnp.concatenate(
        [edge_index[0], loops, jnp.zeros((pad,), i32)])
    dst = jnp.concatenate(
        [edge_index[1], loops, jnp.full((pad,), N, i32)])

    aall0 = _make_aall(as0, ad0)
    aall1 = _make_aall(as1, ad1)
    mavg = (jnp.arange(F)[:, None] % HID ==
            jnp.arange(HID)[None, :]).astype(f32) / HEADS

    b0r = b0.reshape(1, F)
    b1r = b1.reshape(1, HID)
    r8 = (jnp.arange(HEADS)[:, None] ==
          jnp.arange(F)[None, :] // HID).astype(f32)

    h0, asd0, cm0 = _tc_embed(x, W0, aall0)
    op0, dp0 = _sc_layer(h0, asd0, cm0, src, dst)
    s1, s2 = _tc_stat(op0, dp0, r8, b0r)
    h1, asd1, cm1 = _tc_main1(op0, dp0, r8, b0r, g0.reshape(1, F),
                              be0.reshape(1, F), s1, s2, W1, aall1)
    op1, dp1 = _sc_layer(h1, asd1, cm1, src, dst)

    s1f, s2f = _tc_statf(op1, dp1, r8, mavg, b1r)
    return _tc_final(op1, dp1, r8, mavg, b1r, g1.reshape(1, HID),
                     be1.reshape(1, HID), s1f, s2f, Wc, bc.reshape(1, 2))


# speedup vs baseline: 45.0183x; 1.1816x over previous
import functools

import jax
import jax.numpy as jnp
from jax import lax
from jax.experimental import pallas as pl
from jax.experimental.pallas import tpu as pltpu
from jax.experimental.pallas import tpu_sc as plsc

N = 10000
E = 320000
IN = 128
HID = 16
HEADS = 8
F = HEADS * HID
NEG = 0.2

NC = 2
NS = 16
L = 16
NW = NC * NS

EPW = 10368
ET = EPW * NW
NP = 10112
RPT = NP // NS
C1 = 128
C2 = 128
NCH1 = EPW // C1
NCH2 = EPW // C2

f32 = jnp.float32
i32 = jnp.int32


_RB = 400


def _embed_body(x_ref, w_ref, a_ref, h_ref, asd_ref, cm_ref, mx_ref):
    i = pl.program_id(0)
    h = jnp.dot(x_ref[...], w_ref[...], preferred_element_type=f32)
    h_ref[...] = h
    asd = jnp.dot(h, a_ref[...], preferred_element_type=f32)
    asd_ref[...] = asd
    bm = jnp.max(asd, axis=0, keepdims=True)

    @pl.when(i == 0)
    def _():
        mx_ref[...] = bm

    @pl.when(i > 0)
    def _():
        mx_ref[...] = jnp.maximum(mx_ref[...], bm)

    @pl.when(i == pl.num_programs(0) - 1)
    def _():
        m = mx_ref[...]
        c = m[:, :HEADS] + m[:, HEADS:]
        cl = jnp.maximum(c, NEG * c)
        cm_ref[...] = jnp.concatenate([cl, jnp.zeros_like(cl)], axis=1)


def _tc_embed(x, w, aall):
    return pl.pallas_call(
        _embed_body,
        grid=(N // _RB,),
        in_specs=[
            pl.BlockSpec((_RB, IN), lambda i: (i, 0)),
            pl.BlockSpec((IN, F), lambda i: (0, 0)),
            pl.BlockSpec((F, 2 * HEADS), lambda i: (0, 0)),
        ],
        out_specs=[
            pl.BlockSpec((_RB, F), lambda i: (i, 0)),
            pl.BlockSpec((_RB, 2 * HEADS), lambda i: (i, 0)),
            pl.BlockSpec((1, 2 * HEADS), lambda i: (0, 0)),
        ],
        out_shape=[
            jax.ShapeDtypeStruct((N, F), f32),
            jax.ShapeDtypeStruct((N, 2 * HEADS), f32),
            jax.ShapeDtypeStruct((1, 2 * HEADS), f32),
        ],
        scratch_shapes=[pltpu.VMEM((1, 2 * HEADS), f32)],
    )(x, w, aall)


def _stat_body(op_ref, dp_ref, r8_ref, b_ref, s1_ref, s2_ref):
    inv = 1.0 / (dp_ref[0, :N, :] + dp_ref[1, :N, :] + 1e-16)
    scale = jnp.dot(inv, r8_ref[...], preferred_element_type=f32)
    o = (op_ref[0, :N, :] + op_ref[1, :N, :]) * scale + b_ref[...]
    s1_ref[...] = jnp.sum(o, axis=0, keepdims=True)
    s2_ref[...] = jnp.sum(o * o, axis=0, keepdims=True)


def _tc_stat(op, dp, r8, b):
    d = b.shape[1]
    return pl.pallas_call(
        _stat_body,
        out_shape=[
            jax.ShapeDtypeStruct((1, d), f32),
            jax.ShapeDtypeStruct((1, d), f32),
        ],
    )(op, dp, r8, b)


def _main1_body(op_ref, dp_ref, r8_ref, b_ref, g_ref, be_ref, s1_ref, s2_ref,
                w_ref, a_ref, h_ref, asd_ref, cm_ref, mx_ref):
    i = pl.program_id(0)
    inv = 1.0 / (dp_ref[0] + dp_ref[1] + 1e-16)
    scale = jnp.dot(inv, r8_ref[...], preferred_element_type=f32)
    o = (op_ref[0] + op_ref[1]) * scale + b_ref[...]
    m = s1_ref[...] / N
    v = s2_ref[...] / N - m * m
    xn = g_ref[...] * (o - m) * lax.rsqrt(v + 1e-5) + be_ref[...]
    el = jnp.where(xn > 0, xn, jnp.exp(xn) - 1.0)
    h = jnp.dot(el, w_ref[...], preferred_element_type=f32)
    h_ref[...] = h
    asd = jnp.dot(h, a_ref[...], preferred_element_type=f32)
    asd_ref[...] = asd
    bm = jnp.max(asd, axis=0, keepdims=True)

    @pl.when(i == 0)
    def _():
        mx_ref[...] = bm

    @pl.when(i > 0)
    def _():
        mx_ref[...] = jnp.maximum(mx_ref[...], bm)

    @pl.when(i == pl.num_programs(0) - 1)
    def _():
        mm = mx_ref[...]
        c = mm[:, :HEADS] + mm[:, HEADS:]
        cl = jnp.maximum(c, NEG * c)
        cm_ref[...] = jnp.concatenate([cl, jnp.zeros_like(cl)], axis=1)


def _tc_main1(op, dp, r8, b, g, be, s1, s2, w, aall):
    return pl.pallas_call(
        _main1_body,
        grid=(N // _RB,),
        in_specs=[
            pl.BlockSpec((2, _RB, F), lambda i: (0, i, 0)),
            pl.BlockSpec((2, _RB, HEADS), lambda i: (0, i, 0)),
            pl.BlockSpec((HEADS, F), lambda i: (0, 0)),
            pl.BlockSpec((1, F), lambda i: (0, 0)),
            pl.BlockSpec((1, F), lambda i: (0, 0)),
            pl.BlockSpec((1, F), lambda i: (0, 0)),
            pl.BlockSpec((1, F), lambda i: (0, 0)),
            pl.BlockSpec((1, F), lambda i: (0, 0)),
            pl.BlockSpec((F, F), lambda i: (0, 0)),
            pl.BlockSpec((F, 2 * HEADS), lambda i: (0, 0)),
        ],
        out_specs=[
            pl.BlockSpec((_RB, F), lambda i: (i, 0)),
            pl.BlockSpec((_RB, 2 * HEADS), lambda i: (i, 0)),
            pl.BlockSpec((1, 2 * HEADS), lambda i: (0, 0)),
        ],
        out_shape=[
            jax.ShapeDtypeStruct((N, F), f32),
            jax.ShapeDtypeStruct((N, 2 * HEADS), f32),
            jax.ShapeDtypeStruct((1, 2 * HEADS), f32),
        ],
        scratch_shapes=[pltpu.VMEM((1, 2 * HEADS), f32)],
    )(op, dp, r8, b, g, be, s1, s2, w, aall)


def _statf_body(op_ref, dp_ref, r8_ref, mavg_ref, b_ref, s1_ref, s2_ref):
    inv = 1.0 / (dp_ref[0, :N, :] + dp_ref[1, :N, :] + 1e-16)
    scale = jnp.dot(inv, r8_ref[...], preferred_element_type=f32)
    o = (op_ref[0, :N, :] + op_ref[1, :N, :]) * scale
    hm = jnp.dot(o, mavg_ref[...], preferred_element_type=f32) + b_ref[...]
    s1_ref[...] = jnp.sum(hm, axis=0, keepdims=True)
    s2_ref[...] = jnp.sum(hm * hm, axis=0, keepdims=True)


def _tc_statf(op, dp, r8, mavg, b):
    return pl.pallas_call(
        _statf_body,
        out_shape=[
            jax.ShapeDtypeStruct((1, HID), f32),
            jax.ShapeDtypeStruct((1, HID), f32),
        ],
    )(op, dp, r8, mavg, b)


def _final_body(op_ref, dp_ref, r8_ref, mavg_ref, b_ref, g_ref, be_ref,
                s1_ref, s2_ref, wc_ref, bc_ref, out_ref):
    inv = 1.0 / (dp_ref[0, :N, :] + dp_ref[1, :N, :] + 1e-16)
    scale = jnp.dot(inv, r8_ref[...], preferred_element_type=f32)
    o = (op_ref[0, :N, :] + op_ref[1, :N, :]) * scale
    hm = jnp.dot(o, mavg_ref[...], preferred_element_type=f32) + b_ref[...]
    m = s1_ref[...] / N
    v = s2_ref[...] / N - m * m
    xn = g_ref[...] * (hm - m) * lax.rsqrt(v + 1e-5) + be_ref[...]
    out_ref[...] = jnp.dot(xn, wc_ref[...], preferred_element_type=f32) + bc_ref[...]


def _tc_final(op, dp, r8, mavg, b, g, be, s1, s2, wc, bc):
    return pl.pallas_call(
        _final_body,
        out_shape=jax.ShapeDtypeStruct((N, 2), f32),
    )(op, dp, r8, mavg, b, g, be, s1, s2, wc, bc)


_MESH = plsc.VectorSubcoreMesh(
    core_axis_name="c", subcore_axis_name="s", num_cores=NC, num_subcores=NS)


def _sc_pass1_body(src_h, dst_h, alo_h, ahi_h, dlo_h, dhi_h, cm_h,
                   exa_h, exb_h, dp_h,
                   atab, dtab, sidx, didx, exbuf, cm_v, den_sh):
    c = lax.axis_index("c")
    s = lax.axis_index("s")
    wid = c * NS + s
    ebase = wid * EPW
    iota = lax.iota(i32, L)
    zero = jnp.zeros((L,), f32)

    pltpu.sync_copy(cm_h, cm_v)
    cmvec = cm_v[...]

    for k in range(C1 * HEADS // L):
        flat = iota + k * L
        plsc.store_scatter(exbuf, [flat // HEADS, flat % HEADS], zero)
    for r in range(0, RPT, C1):
        rows = min(C1, RPT - r)
        pltpu.sync_copy(exbuf.at[pl.ds(0, rows), :],
                        den_sh.at[pl.ds(s * RPT + r, rows), :])
    plsc.subcore_barrier()

    for phase in range(2):
        if phase == 0:
            pltpu.sync_copy(alo_h, atab)
            pltpu.sync_copy(dlo_h, dtab)
        else:
            for k in range(C1 * HEADS // L):
                flat = iota + k * L
                plsc.store_scatter(exbuf, [flat // HEADS, flat % HEADS], zero)
            pltpu.sync_copy(ahi_h, atab)
            pltpu.sync_copy(dhi_h, dtab)
        col0 = phase * 4
        ex_out = exa_h if phase == 0 else exb_h

        def chunk_body(ci, carry):
            base = ebase + ci * C1
            pltpu.sync_copy(src_h.at[pl.ds(base, C1)], sidx)
            pltpu.sync_copy(dst_h.at[pl.ds(base, C1)], didx)
            for g in range(C1 // L):
                sv = sidx[pl.ds(g * L, L)]
                dv = didx[pl.ds(g * L, L)]
                ri = iota + g * L
                for hh in range(4):
                    av = plsc.load_gather(atab, [sv * 4 + hh])
                    bv = plsc.load_gather(dtab, [dv * 4 + hh])
                    e = av + bv
                    e = jnp.maximum(e, NEG * e)
                    ex = jnp.exp(e - cmvec[col0 + hh])
                    plsc.store_scatter(
                        exbuf, [ri, jnp.full((L,), col0 + hh, i32)], ex)
            pltpu.sync_copy(exbuf, ex_out.at[pl.ds(base, C1), :])
            pltpu.sync_copy(exbuf, den_sh.at[didx], add=True)
            return carry

        lax.fori_loop(0, NCH1, chunk_body, 0)

    plsc.subcore_barrier()
    pltpu.sync_copy(den_sh.at[pl.ds(s * RPT, RPT), :],
                    dp_h.at[c, pl.ds(s * RPT, RPT), :])


_sc_pass1 = functools.partial(
    pl.kernel,
    out_type=(
        jax.ShapeDtypeStruct((ET, HEADS), f32),
        jax.ShapeDtypeStruct((ET, HEADS), f32),
        jax.ShapeDtypeStruct((NC, NP, HEADS), f32),
    ),
    mesh=_MESH,
    compiler_params=pltpu.CompilerParams(
        needs_layout_passes=False, use_tc_tiling_on_sc=False),
    scratch_types=[
        pltpu.VMEM((N * 4,), f32),
        pltpu.VMEM((N * 4,), f32),
        pltpu.VMEM((C1,), i32),
        pltpu.VMEM((C1,), i32),
        pltpu.VMEM((C1, HEADS), f32),
        pltpu.VMEM((L,), f32),
        pltpu.VMEM_SHARED((NP, HEADS), f32),
    ],
)(_sc_pass1_body)


def _sc_pass2_body(src_h, dst_h, h_hbm, exa_h, exb_h,
                   op_h,
                   sidx, didx, exba, exbb, alphab, hrows, msgb,
                   sem, out_sh):
    c = lax.axis_index("c")
    s = lax.axis_index("s")
    wid = c * NS + s
    ebase = wid * EPW
    iota = lax.iota(i32, L)
    zero = jnp.zeros((L,), f32)

    for r in range(C2):
        for g in range(F // L):
            msgb[r, pl.ds(g * L, L)] = zero
    for r in range(0, RPT, C2):
        rows = min(C2, RPT - r)
        pltpu.sync_copy(msgb.at[pl.ds(0, rows), :],
                        out_sh.at[pl.ds(s * RPT + r, rows), :])
    plsc.subcore_barrier()

    def chunk_body(ci, carry):
        base = ebase + ci * C2
        pltpu.sync_copy(src_h.at[pl.ds(base, C2)], sidx)
        pltpu.sync_copy(dst_h.at[pl.ds(base, C2)], didx)
        pltpu.sync_copy(exa_h.at[pl.ds(base, C2), :], exba)
        pltpu.sync_copy(exb_h.at[pl.ds(base, C2), :], exbb)
        pltpu.async_copy(h_hbm.at[sidx], hrows, sem).wait()
        for g in range(C2 // L):
            rv = iota + g * L
            for h in range(HEADS):
                hsp = jnp.full((L,), h, i32)
                exv = (plsc.load_gather(exba, [rv, hsp]) +
                       plsc.load_gather(exbb, [rv, hsp]))
                plsc.store_scatter(alphab, [rv, hsp], exv)

        def edge_body(e, cc):
            arow = alphab[e, :]
            for g in range(HEADS):
                msgb[e, pl.ds(g * HID, HID)] = (
                    arow[g] * hrows[e, pl.ds(g * HID, HID)])
            return cc

        lax.fori_loop(0, C2, edge_body, 0)
        pltpu.sync_copy(msgb, out_sh.at[didx], add=True)
        return carry

    lax.fori_loop(0, NCH2, chunk_body, 0)

    plsc.subcore_barrier()
    pltpu.sync_copy(out_sh.at[pl.ds(s * RPT, RPT), :],
                    op_h.at[c, pl.ds(s * RPT, RPT), :])


_sc_pass2 = functools.partial(
    pl.kernel,
    out_type=jax.ShapeDtypeStruct((NC, NP, F), f32),
    mesh=_MESH,
    compiler_params=pltpu.CompilerParams(
        needs_layout_passes=False, use_tc_tiling_on_sc=False),
    scratch_types=[
        pltpu.VMEM((C2,), i32),
        pltpu.VMEM((C2,), i32),
        pltpu.VMEM((C2, HEADS), f32),
        pltpu.VMEM((C2, HEADS), f32),
        pltpu.VMEM((C2, L), f32),
        pltpu.VMEM((C2, F), f32),
        pltpu.VMEM((C2, F), f32),
        pltpu.SemaphoreType.DMA,
        pltpu.VMEM_SHARED((NP, F), f32),
    ],
)(_sc_pass2_body)


def _make_aall(a_src, a_dst):
    jj = jnp.arange(F)
    blk = (jj[:, None] // HID == jnp.arange(HEADS)[None, :]).astype(f32)
    return jnp.concatenate(
        [blk * a_src.reshape(F)[:, None], blk * a_dst.reshape(F)[:, None]],
        axis=1)


def _sc_layer(h, asd, cm, src, dst):
    alo = asd[:, 0:4].reshape(-1)
    ahi = asd[:, 4:8].reshape(-1)
    dlo = asd[:, 8:12].reshape(-1)
    dhi = asd[:, 12:16].reshape(-1)
    cmv = cm.reshape(2 * HEADS)
    exa, exb, dp = _sc_pass1(src, dst, alo, ahi, dlo, dhi, cmv)
    return _sc_pass2(src, dst, h, exa, exb), dp


def kernel(x, edge_index, W0, as0, ad0, b0, g0, be0,
           W1, as1, ad1, b1, g1, be1, Wc, bc):
    pad = ET - E - N
    loops = jnp.arange(N, dtype=i32)
    src = jnp.concatenate(
        [edge_index[0], loops, jnp.zeros((pad,), i32)])
    dst = jnp.concatenate(
        [edge_index[1], loops, jnp.full((pad,), N, i32)])

    aall0 = _make_aall(as0, ad0)
    aall1 = _make_aall(as1, ad1)
    mavg = (jnp.arange(F)[:, None] % HID ==
            jnp.arange(HID)[None, :]).astype(f32) / HEADS

    b0r = b0.reshape(1, F)
    b1r = b1.reshape(1, HID)
    r8 = (jnp.arange(HEADS)[:, None] ==
          jnp.arange(F)[None, :] // HID).astype(f32)

    h0, asd0, cm0 = _tc_embed(x, W0, aall0)
    op0, dp0 = _sc_layer(h0, asd0, cm0, src, dst)
    s1, s2 = _tc_stat(op0, dp0, r8, b0r)
    h1, asd1, cm1 = _tc_main1(op0, dp0, r8, b0r, g0.reshape(1, F),
                              be0.reshape(1, F), s1, s2, W1, aall1)
    op1, dp1 = _sc_layer(h1, asd1, cm1, src, dst)

    s1f, s2f = _tc_statf(op1, dp1, r8, mavg, b1r)
    return _tc_final(op1, dp1, r8, mavg, b1r, g1.reshape(1, HID),
                     be1.reshape(1, HID), s1f, s2f, Wc, bc.reshape(1, 2))


# speedup vs baseline: 60.0820x; 1.3346x over previous
import functools

import jax
import jax.numpy as jnp
from jax import lax
from jax.experimental import pallas as pl
from jax.experimental.pallas import tpu as pltpu
from jax.experimental.pallas import tpu_sc as plsc

N = 10000
E = 320000
IN = 128
HID = 16
HEADS = 8
F = HEADS * HID
NEG = 0.2

NC = 2
NS = 16
L = 16
NW = NC * NS

EPW = 10368
ET = EPW * NW
NP = 10112
RPT = NP // NS
C1 = 128
C2 = 128
NCH1 = EPW // C1
NCH2 = EPW // C2

f32 = jnp.float32
i32 = jnp.int32


_RB = 400


def _embed_body(x_ref, w_ref, a_ref, h_ref, asd_ref, cm_ref, mx_ref):
    i = pl.program_id(0)
    h = jnp.dot(x_ref[...], w_ref[...], preferred_element_type=f32)
    h_ref[...] = h
    asd = jnp.dot(h, a_ref[...], preferred_element_type=f32)
    asd_ref[...] = asd
    bm = jnp.max(asd, axis=0, keepdims=True)

    @pl.when(i == 0)
    def _():
        mx_ref[...] = bm

    @pl.when(i > 0)
    def _():
        mx_ref[...] = jnp.maximum(mx_ref[...], bm)

    @pl.when(i == pl.num_programs(0) - 1)
    def _():
        m = mx_ref[...]
        c = m[:, :HEADS] + m[:, HEADS:]
        cl = jnp.maximum(c, NEG * c)
        cm_ref[...] = jnp.concatenate([cl, jnp.zeros_like(cl)], axis=1)


def _tc_embed(x, w, aall):
    return pl.pallas_call(
        _embed_body,
        grid=(N // _RB,),
        in_specs=[
            pl.BlockSpec((_RB, IN), lambda i: (i, 0)),
            pl.BlockSpec((IN, F), lambda i: (0, 0)),
            pl.BlockSpec((F, 2 * HEADS), lambda i: (0, 0)),
        ],
        out_specs=[
            pl.BlockSpec((_RB, F), lambda i: (i, 0)),
            pl.BlockSpec((_RB, 2 * HEADS), lambda i: (i, 0)),
            pl.BlockSpec((1, 2 * HEADS), lambda i: (0, 0)),
        ],
        out_shape=[
            jax.ShapeDtypeStruct((N, F), f32),
            jax.ShapeDtypeStruct((N, 2 * HEADS), f32),
            jax.ShapeDtypeStruct((1, 2 * HEADS), f32),
        ],
        scratch_shapes=[pltpu.VMEM((1, 2 * HEADS), f32)],
    )(x, w, aall)


def _stat_body(op_ref, dp_ref, r8_ref, b_ref, s1_ref, s2_ref):
    inv = 1.0 / (dp_ref[0, :N, :] + dp_ref[1, :N, :] + 1e-16)
    scale = jnp.dot(inv, r8_ref[...], preferred_element_type=f32)
    o = (op_ref[0, :N, :] + op_ref[1, :N, :]) * scale + b_ref[...]
    s1_ref[...] = jnp.sum(o, axis=0, keepdims=True)
    s2_ref[...] = jnp.sum(o * o, axis=0, keepdims=True)


def _tc_stat(op, dp, r8, b):
    d = b.shape[1]
    return pl.pallas_call(
        _stat_body,
        out_shape=[
            jax.ShapeDtypeStruct((1, d), f32),
            jax.ShapeDtypeStruct((1, d), f32),
        ],
    )(op, dp, r8, b)


def _main1_body(op_ref, dp_ref, r8_ref, b_ref, g_ref, be_ref, s1_ref, s2_ref,
                w_ref, a_ref, h_ref, asd_ref, cm_ref, mx_ref):
    i = pl.program_id(0)
    inv = 1.0 / (dp_ref[0] + dp_ref[1] + 1e-16)
    scale = jnp.dot(inv, r8_ref[...], preferred_element_type=f32)
    o = (op_ref[0] + op_ref[1]) * scale + b_ref[...]
    m = s1_ref[...] / N
    v = s2_ref[...] / N - m * m
    xn = g_ref[...] * (o - m) * lax.rsqrt(v + 1e-5) + be_ref[...]
    el = jnp.where(xn > 0, xn, jnp.exp(xn) - 1.0)
    h = jnp.dot(el, w_ref[...], preferred_element_type=f32)
    h_ref[...] = h
    asd = jnp.dot(h, a_ref[...], preferred_element_type=f32)
    asd_ref[...] = asd
    bm = jnp.max(asd, axis=0, keepdims=True)

    @pl.when(i == 0)
    def _():
        mx_ref[...] = bm

    @pl.when(i > 0)
    def _():
        mx_ref[...] = jnp.maximum(mx_ref[...], bm)

    @pl.when(i == pl.num_programs(0) - 1)
    def _():
        mm = mx_ref[...]
        c = mm[:, :HEADS] + mm[:, HEADS:]
        cl = jnp.maximum(c, NEG * c)
        cm_ref[...] = jnp.concatenate([cl, jnp.zeros_like(cl)], axis=1)


def _tc_main1(op, dp, r8, b, g, be, s1, s2, w, aall):
    return pl.pallas_call(
        _main1_body,
        grid=(N // _RB,),
        in_specs=[
            pl.BlockSpec((2, _RB, F), lambda i: (0, i, 0)),
            pl.BlockSpec((2, _RB, HEADS), lambda i: (0, i, 0)),
            pl.BlockSpec((HEADS, F), lambda i: (0, 0)),
            pl.BlockSpec((1, F), lambda i: (0, 0)),
            pl.BlockSpec((1, F), lambda i: (0, 0)),
            pl.BlockSpec((1, F), lambda i: (0, 0)),
            pl.BlockSpec((1, F), lambda i: (0, 0)),
            pl.BlockSpec((1, F), lambda i: (0, 0)),
            pl.BlockSpec((F, F), lambda i: (0, 0)),
            pl.BlockSpec((F, 2 * HEADS), lambda i: (0, 0)),
        ],
        out_specs=[
            pl.BlockSpec((_RB, F), lambda i: (i, 0)),
            pl.BlockSpec((_RB, 2 * HEADS), lambda i: (i, 0)),
            pl.BlockSpec((1, 2 * HEADS), lambda i: (0, 0)),
        ],
        out_shape=[
            jax.ShapeDtypeStruct((N, F), f32),
            jax.ShapeDtypeStruct((N, 2 * HEADS), f32),
            jax.ShapeDtypeStruct((1, 2 * HEADS), f32),
        ],
        scratch_shapes=[pltpu.VMEM((1, 2 * HEADS), f32)],
    )(op, dp, r8, b, g, be, s1, s2, w, aall)


def _statf_body(op_ref, dp_ref, r8_ref, mavg_ref, b_ref, s1_ref, s2_ref):
    inv = 1.0 / (dp_ref[0, :N, :] + dp_ref[1, :N, :] + 1e-16)
    scale = jnp.dot(inv, r8_ref[...], preferred_element_type=f32)
    o = (op_ref[0, :N, :] + op_ref[1, :N, :]) * scale
    hm = jnp.dot(o, mavg_ref[...], preferred_element_type=f32) + b_ref[...]
    s1_ref[...] = jnp.sum(hm, axis=0, keepdims=True)
    s2_ref[...] = jnp.sum(hm * hm, axis=0, keepdims=True)


def _tc_statf(op, dp, r8, mavg, b):
    return pl.pallas_call(
        _statf_body,
        out_shape=[
            jax.ShapeDtypeStruct((1, HID), f32),
            jax.ShapeDtypeStruct((1, HID), f32),
        ],
    )(op, dp, r8, mavg, b)


def _final_body(op_ref, dp_ref, r8_ref, mavg_ref, b_ref, g_ref, be_ref,
                s1_ref, s2_ref, wc_ref, bc_ref, out_ref):
    inv = 1.0 / (dp_ref[0, :N, :] + dp_ref[1, :N, :] + 1e-16)
    scale = jnp.dot(inv, r8_ref[...], preferred_element_type=f32)
    o = (op_ref[0, :N, :] + op_ref[1, :N, :]) * scale
    hm = jnp.dot(o, mavg_ref[...], preferred_element_type=f32) + b_ref[...]
    m = s1_ref[...] / N
    v = s2_ref[...] / N - m * m
    xn = g_ref[...] * (hm - m) * lax.rsqrt(v + 1e-5) + be_ref[...]
    out_ref[...] = jnp.dot(xn, wc_ref[...], preferred_element_type=f32) + bc_ref[...]


def _tc_final(op, dp, r8, mavg, b, g, be, s1, s2, wc, bc):
    return pl.pallas_call(
        _final_body,
        out_shape=jax.ShapeDtypeStruct((N, 2), f32),
    )(op, dp, r8, mavg, b, g, be, s1, s2, wc, bc)


_MESH = plsc.VectorSubcoreMesh(
    core_axis_name="c", subcore_axis_name="s", num_cores=NC, num_subcores=NS)


def _sc_pass1_body(src_h, dst_h, alo_h, ahi_h, dlo_h, dhi_h, cm_h,
                   exa_h, exb_h, dp_h,
                   atab, dtab, sidx, didx, exbuf, cm_v, sem, sem2, den_sh):
    c = lax.axis_index("c")
    s = lax.axis_index("s")
    wid = c * NS + s
    ebase = wid * EPW
    iota = lax.iota(i32, L)
    zero = jnp.zeros((L,), f32)

    pltpu.sync_copy(cm_h, cm_v)
    cmvec = cm_v[...]

    for k in range(C1 * HEADS // L):
        flat = iota + k * L
        plsc.store_scatter(exbuf, [flat // HEADS, flat % HEADS], zero)
    for r in range(0, RPT, C1):
        rows = min(C1, RPT - r)
        pltpu.sync_copy(exbuf.at[pl.ds(0, rows), :],
                        den_sh.at[pl.ds(s * RPT + r, rows), :])
    plsc.subcore_barrier()

    for phase in range(2):
        if phase == 0:
            pltpu.sync_copy(alo_h, atab)
            pltpu.sync_copy(dlo_h, dtab)
        else:
            for k in range(C1 * HEADS // L):
                flat = iota + k * L
                plsc.store_scatter(exbuf, [flat // HEADS, flat % HEADS], zero)
            pltpu.sync_copy(ahi_h, atab)
            pltpu.sync_copy(dhi_h, dtab)
        col0 = phase * 4
        ex_out = exa_h if phase == 0 else exb_h

        def chunk_body(ci, carry):
            base = ebase + ci * C1
            d1 = pltpu.async_copy(src_h.at[pl.ds(base, C1)], sidx, sem)
            d2 = pltpu.async_copy(dst_h.at[pl.ds(base, C1)], didx, sem)
            d1.wait()
            d2.wait()
            for g in range(C1 // L):
                sv = sidx[pl.ds(g * L, L)]
                dv = didx[pl.ds(g * L, L)]
                ri = iota + g * L
                for hh in range(4):
                    av = plsc.load_gather(atab, [sv * 4 + hh])
                    bv = plsc.load_gather(dtab, [dv * 4 + hh])
                    e = av + bv
                    e = jnp.maximum(e, NEG * e)
                    ex = jnp.exp(e - cmvec[col0 + hh])
                    plsc.store_scatter(
                        exbuf, [ri, jnp.full((L,), col0 + hh, i32)], ex)
            e1 = pltpu.async_copy(exbuf, ex_out.at[pl.ds(base, C1), :], sem)
            e2 = pltpu.async_copy(exbuf, den_sh.at[didx], sem2, add=True)
            e1.wait()
            e2.wait()
            return carry

        lax.fori_loop(0, NCH1, chunk_body, 0)

    plsc.subcore_barrier()
    pltpu.sync_copy(den_sh.at[pl.ds(s * RPT, RPT), :],
                    dp_h.at[c, pl.ds(s * RPT, RPT), :])


_sc_pass1 = functools.partial(
    pl.kernel,
    out_type=(
        jax.ShapeDtypeStruct((ET, HEADS), f32),
        jax.ShapeDtypeStruct((ET, HEADS), f32),
        jax.ShapeDtypeStruct((NC, NP, HEADS), f32),
    ),
    mesh=_MESH,
    compiler_params=pltpu.CompilerParams(
        needs_layout_passes=False, use_tc_tiling_on_sc=False),
    scratch_types=[
        pltpu.VMEM((N * 4,), f32),
        pltpu.VMEM((N * 4,), f32),
        pltpu.VMEM((C1,), i32),
        pltpu.VMEM((C1,), i32),
        pltpu.VMEM((C1, HEADS), f32),
        pltpu.VMEM((L,), f32),
        pltpu.SemaphoreType.DMA,
        pltpu.SemaphoreType.DMA,
        pltpu.VMEM_SHARED((NP, HEADS), f32),
    ],
)(_sc_pass1_body)


def _sc_pass2_body(src_h, dst_h, h_hbm, exa_h, exb_h,
                   op_h,
                   sidx, didx, exba, exbb, alphab, hrows, msgb,
                   sem, semg, sem2, out_sh):
    c = lax.axis_index("c")
    s = lax.axis_index("s")
    wid = c * NS + s
    ebase = wid * EPW
    iota = lax.iota(i32, L)
    zero = jnp.zeros((L,), f32)

    for r in range(C2):
        for g in range(F // L):
            msgb[r, pl.ds(g * L, L)] = zero
    for r in range(0, RPT, C2):
        rows = min(C2, RPT - r)
        pltpu.sync_copy(msgb.at[pl.ds(0, rows), :],
                        out_sh.at[pl.ds(s * RPT + r, rows), :])
    plsc.subcore_barrier()

    def chunk_body(ci, carry):
        base = ebase + ci * C2
        d1 = pltpu.async_copy(src_h.at[pl.ds(base, C2)], sidx, sem)
        d2 = pltpu.async_copy(dst_h.at[pl.ds(base, C2)], didx, sem)
        d3 = pltpu.async_copy(exa_h.at[pl.ds(base, C2), :], exba, sem)
        d4 = pltpu.async_copy(exb_h.at[pl.ds(base, C2), :], exbb, sem)
        d1.wait()
        g1 = pltpu.async_copy(h_hbm.at[sidx], hrows, semg)
        d2.wait()
        d3.wait()
        d4.wait()
        for g in range(C2 // L):
            rv = iota + g * L
            for h in range(HEADS):
                hsp = jnp.full((L,), h, i32)
                exv = (plsc.load_gather(exba, [rv, hsp]) +
                       plsc.load_gather(exbb, [rv, hsp]))
                plsc.store_scatter(alphab, [rv, hsp], exv)
        g1.wait()

        def edge_body(e, cc):
            arow = alphab[e, :]
            for g in range(HEADS):
                msgb[e, pl.ds(g * HID, HID)] = (
                    arow[g] * hrows[e, pl.ds(g * HID, HID)])
            return cc

        lax.fori_loop(0, C2, edge_body, 0)
        pltpu.async_copy(msgb, out_sh.at[didx], sem2, add=True).wait()
        return carry

    lax.fori_loop(0, NCH2, chunk_body, 0)

    plsc.subcore_barrier()
    pltpu.sync_copy(out_sh.at[pl.ds(s * RPT, RPT), :],
                    op_h.at[c, pl.ds(s * RPT, RPT), :])


_sc_pass2 = functools.partial(
    pl.kernel,
    out_type=jax.ShapeDtypeStruct((NC, NP, F), f32),
    mesh=_MESH,
    compiler_params=pltpu.CompilerParams(
        needs_layout_passes=False, use_tc_tiling_on_sc=False),
    scratch_types=[
        pltpu.VMEM((C2,), i32),
        pltpu.VMEM((C2,), i32),
        pltpu.VMEM((C2, HEADS), f32),
        pltpu.VMEM((C2, HEADS), f32),
        pltpu.VMEM((C2, L), f32),
        pltpu.VMEM((C2, F), f32),
        pltpu.VMEM((C2, F), f32),
        pltpu.SemaphoreType.DMA,
        pltpu.SemaphoreType.DMA,
        pltpu.SemaphoreType.DMA,
        pltpu.VMEM_SHARED((NP, F), f32),
    ],
)(_sc_pass2_body)


def _make_aall(a_src, a_dst):
    jj = jnp.arange(F)
    blk = (jj[:, None] // HID == jnp.arange(HEADS)[None, :]).astype(f32)
    return jnp.concatenate(
        [blk * a_src.reshape(F)[:, None], blk * a_dst.reshape(F)[:, None]],
        axis=1)


def _sc_layer(h, asd, cm, src, dst):
    alo = asd[:, 0:4].reshape(-1)
    ahi = asd[:, 4:8].reshape(-1)
    dlo = asd[:, 8:12].reshape(-1)
    dhi = asd[:, 12:16].reshape(-1)
    cmv = cm.reshape(2 * HEADS)
    exa, exb, dp = _sc_pass1(src, dst, alo, ahi, dlo, dhi, cmv)
    return _sc_pass2(src, dst, h, exa, exb), dp


def kernel(x, edge_index, W0, as0, ad0, b0, g0, be0,
           W1, as1, ad1, b1, g1, be1, Wc, bc):
    pad = ET - E - N
    loops = jnp.arange(N, dtype=i32)
    src = jnp.concatenate(
        [edge_index[0], loops, jnp.zeros((pad,), i32)])
    dst = jnp.concatenate(
        [edge_index[1], loops, jnp.full((pad,), N, i32)])

    aall0 = _make_aall(as0, ad0)
    aall1 = _make_aall(as1, ad1)
    mavg = (jnp.arange(F)[:, None] % HID ==
            jnp.arange(HID)[None, :]).astype(f32) / HEADS

    b0r = b0.reshape(1, F)
    b1r = b1.reshape(1, HID)
    r8 = (jnp.arange(HEADS)[:, None] ==
          jnp.arange(F)[None, :] // HID).astype(f32)

    h0, asd0, cm0 = _tc_embed(x, W0, aall0)
    op0, dp0 = _sc_layer(h0, asd0, cm0, src, dst)
    s1, s2 = _tc_stat(op0, dp0, r8, b0r)
    h1, asd1, cm1 = _tc_main1(op0, dp0, r8, b0r, g0.reshape(1, F),
                              be0.reshape(1, F), s1, s2, W1, aall1)
    op1, dp1 = _sc_layer(h1, asd1, cm1, src, dst)

    s1f, s2f = _tc_statf(op1, dp1, r8, mavg, b1r)
    return _tc_final(op1, dp1, r8, mavg, b1r, g1.reshape(1, HID),
                     be1.reshape(1, HID), s1f, s2f, Wc, bc.reshape(1, 2))


# speedup vs baseline: 63.1975x; 1.0519x over previous
import functools

import jax
import jax.numpy as jnp
from jax import lax
from jax.experimental import pallas as pl
from jax.experimental.pallas import tpu as pltpu
from jax.experimental.pallas import tpu_sc as plsc

N = 10000
E = 320000
IN = 128
HID = 16
HEADS = 8
F = HEADS * HID
NEG = 0.2

NC = 2
NS = 16
L = 16
NW = NC * NS

EPW = 10368
ET = EPW * NW
NP = 10112
RPT = NP // NS
C1 = 128
C2 = 128
NCH1 = EPW // C1
NCH2 = EPW // C2

f32 = jnp.float32
i32 = jnp.int32


_RB = 400


def _embed_body(x_ref, w_ref, a_ref, h_ref, asd_ref, cm_ref, mx_ref):
    i = pl.program_id(0)
    h = jnp.dot(x_ref[...], w_ref[...], preferred_element_type=f32)
    h_ref[...] = h
    asd = jnp.dot(h, a_ref[...], preferred_element_type=f32)
    asd_ref[...] = asd
    bm = jnp.max(asd, axis=0, keepdims=True)

    @pl.when(i == 0)
    def _():
        mx_ref[...] = bm

    @pl.when(i > 0)
    def _():
        mx_ref[...] = jnp.maximum(mx_ref[...], bm)

    @pl.when(i == pl.num_programs(0) - 1)
    def _():
        m = mx_ref[...]
        c = m[:, :HEADS] + m[:, HEADS:]
        cl = jnp.maximum(c, NEG * c)
        cm_ref[...] = jnp.concatenate([cl, jnp.zeros_like(cl)], axis=1)


def _tc_embed(x, w, aall):
    return pl.pallas_call(
        _embed_body,
        grid=(N // _RB,),
        in_specs=[
            pl.BlockSpec((_RB, IN), lambda i: (i, 0)),
            pl.BlockSpec((IN, F), lambda i: (0, 0)),
            pl.BlockSpec((F, 2 * HEADS), lambda i: (0, 0)),
        ],
        out_specs=[
            pl.BlockSpec((_RB, F), lambda i: (i, 0)),
            pl.BlockSpec((_RB, 2 * HEADS), lambda i: (i, 0)),
            pl.BlockSpec((1, 2 * HEADS), lambda i: (0, 0)),
        ],
        out_shape=[
            jax.ShapeDtypeStruct((N, F), f32),
            jax.ShapeDtypeStruct((N, 2 * HEADS), f32),
            jax.ShapeDtypeStruct((1, 2 * HEADS), f32),
        ],
        scratch_shapes=[pltpu.VMEM((1, 2 * HEADS), f32)],
    )(x, w, aall)


def _stat_body(op_ref, dp_ref, r8_ref, b_ref, s1_ref, s2_ref):
    inv = 1.0 / (dp_ref[0, :N, :] + dp_ref[1, :N, :] + 1e-16)
    scale = jnp.dot(inv, r8_ref[...], preferred_element_type=f32)
    o = (op_ref[0, :N, :] + op_ref[1, :N, :]) * scale + b_ref[...]
    s1_ref[...] = jnp.sum(o, axis=0, keepdims=True)
    s2_ref[...] = jnp.sum(o * o, axis=0, keepdims=True)


def _tc_stat(op, dp, r8, b):
    d = b.shape[1]
    return pl.pallas_call(
        _stat_body,
        out_shape=[
            jax.ShapeDtypeStruct((1, d), f32),
            jax.ShapeDtypeStruct((1, d), f32),
        ],
    )(op, dp, r8, b)


def _main1_body(op_ref, dp_ref, r8_ref, b_ref, g_ref, be_ref, s1_ref, s2_ref,
                w_ref, a_ref, h_ref, asd_ref, cm_ref, mx_ref):
    i = pl.program_id(0)
    inv = 1.0 / (dp_ref[0] + dp_ref[1] + 1e-16)
    scale = jnp.dot(inv, r8_ref[...], preferred_element_type=f32)
    o = (op_ref[0] + op_ref[1]) * scale + b_ref[...]
    m = s1_ref[...] / N
    v = s2_ref[...] / N - m * m
    xn = g_ref[...] * (o - m) * lax.rsqrt(v + 1e-5) + be_ref[...]
    el = jnp.where(xn > 0, xn, jnp.exp(xn) - 1.0)
    h = jnp.dot(el, w_ref[...], preferred_element_type=f32)
    h_ref[...] = h
    asd = jnp.dot(h, a_ref[...], preferred_element_type=f32)
    asd_ref[...] = asd
    bm = jnp.max(asd, axis=0, keepdims=True)

    @pl.when(i == 0)
    def _():
        mx_ref[...] = bm

    @pl.when(i > 0)
    def _():
        mx_ref[...] = jnp.maximum(mx_ref[...], bm)

    @pl.when(i == pl.num_programs(0) - 1)
    def _():
        mm = mx_ref[...]
        c = mm[:, :HEADS] + mm[:, HEADS:]
        cl = jnp.maximum(c, NEG * c)
        cm_ref[...] = jnp.concatenate([cl, jnp.zeros_like(cl)], axis=1)


def _tc_main1(op, dp, r8, b, g, be, s1, s2, w, aall):
    return pl.pallas_call(
        _main1_body,
        grid=(N // _RB,),
        in_specs=[
            pl.BlockSpec((2, _RB, F), lambda i: (0, i, 0)),
            pl.BlockSpec((2, _RB, HEADS), lambda i: (0, i, 0)),
            pl.BlockSpec((HEADS, F), lambda i: (0, 0)),
            pl.BlockSpec((1, F), lambda i: (0, 0)),
            pl.BlockSpec((1, F), lambda i: (0, 0)),
            pl.BlockSpec((1, F), lambda i: (0, 0)),
            pl.BlockSpec((1, F), lambda i: (0, 0)),
            pl.BlockSpec((1, F), lambda i: (0, 0)),
            pl.BlockSpec((F, F), lambda i: (0, 0)),
            pl.BlockSpec((F, 2 * HEADS), lambda i: (0, 0)),
        ],
        out_specs=[
            pl.BlockSpec((_RB, F), lambda i: (i, 0)),
            pl.BlockSpec((_RB, 2 * HEADS), lambda i: (i, 0)),
            pl.BlockSpec((1, 2 * HEADS), lambda i: (0, 0)),
        ],
        out_shape=[
            jax.ShapeDtypeStruct((N, F), f32),
            jax.ShapeDtypeStruct((N, 2 * HEADS), f32),
            jax.ShapeDtypeStruct((1, 2 * HEADS), f32),
        ],
        scratch_shapes=[pltpu.VMEM((1, 2 * HEADS), f32)],
    )(op, dp, r8, b, g, be, s1, s2, w, aall)


def _statf_body(op_ref, dp_ref, r8_ref, mavg_ref, b_ref, s1_ref, s2_ref):
    inv = 1.0 / (dp_ref[0, :N, :] + dp_ref[1, :N, :] + 1e-16)
    scale = jnp.dot(inv, r8_ref[...], preferred_element_type=f32)
    o = (op_ref[0, :N, :] + op_ref[1, :N, :]) * scale
    hm = jnp.dot(o, mavg_ref[...], preferred_element_type=f32) + b_ref[...]
    s1_ref[...] = jnp.sum(hm, axis=0, keepdims=True)
    s2_ref[...] = jnp.sum(hm * hm, axis=0, keepdims=True)


def _tc_statf(op, dp, r8, mavg, b):
    return pl.pallas_call(
        _statf_body,
        out_shape=[
            jax.ShapeDtypeStruct((1, HID), f32),
            jax.ShapeDtypeStruct((1, HID), f32),
        ],
    )(op, dp, r8, mavg, b)


def _final_body(op_ref, dp_ref, r8_ref, mavg_ref, b_ref, g_ref, be_ref,
                s1_ref, s2_ref, wc_ref, bc_ref, out_ref):
    inv = 1.0 / (dp_ref[0, :N, :] + dp_ref[1, :N, :] + 1e-16)
    scale = jnp.dot(inv, r8_ref[...], preferred_element_type=f32)
    o = (op_ref[0, :N, :] + op_ref[1, :N, :]) * scale
    hm = jnp.dot(o, mavg_ref[...], preferred_element_type=f32) + b_ref[...]
    m = s1_ref[...] / N
    v = s2_ref[...] / N - m * m
    xn = g_ref[...] * (hm - m) * lax.rsqrt(v + 1e-5) + be_ref[...]
    out_ref[...] = jnp.dot(xn, wc_ref[...], preferred_element_type=f32) + bc_ref[...]


def _tc_final(op, dp, r8, mavg, b, g, be, s1, s2, wc, bc):
    return pl.pallas_call(
        _final_body,
        out_shape=jax.ShapeDtypeStruct((N, 2), f32),
    )(op, dp, r8, mavg, b, g, be, s1, s2, wc, bc)


_MESH = plsc.VectorSubcoreMesh(
    core_axis_name="c", subcore_axis_name="s", num_cores=NC, num_subcores=NS)


def _sc_pass1_body(src_h, dst_h, alo_h, ahi_h, dlo_h, dhi_h, cm_h,
                   exa_h, exb_h, dp_h,
                   atab, dtab, sidx, didx, exbuf, cm_v, sem, sem2, den_sh):
    c = lax.axis_index("c")
    s = lax.axis_index("s")
    wid = c * NS + s
    ebase = wid * EPW
    iota = lax.iota(i32, L)
    zero = jnp.zeros((L,), f32)

    pltpu.sync_copy(cm_h, cm_v)
    cmvec = cm_v[...]

    for k in range(C1 * HEADS // L):
        flat = iota + k * L
        plsc.store_scatter(exbuf, [flat // HEADS, flat % HEADS], zero)
    for r in range(0, RPT, C1):
        rows = min(C1, RPT - r)
        pltpu.sync_copy(exbuf.at[pl.ds(0, rows), :],
                        den_sh.at[pl.ds(s * RPT + r, rows), :])
    plsc.subcore_barrier()

    for phase in range(2):
        if phase == 0:
            pltpu.sync_copy(alo_h, atab)
            pltpu.sync_copy(dlo_h, dtab)
        else:
            for k in range(C1 * HEADS // L):
                flat = iota + k * L
                plsc.store_scatter(exbuf, [flat // HEADS, flat % HEADS], zero)
            pltpu.sync_copy(ahi_h, atab)
            pltpu.sync_copy(dhi_h, dtab)
        col0 = phase * 4
        ex_out = exa_h if phase == 0 else exb_h

        def chunk_body(ci, carry):
            base = ebase + ci * C1
            d1 = pltpu.async_copy(src_h.at[pl.ds(base, C1)], sidx, sem)
            d2 = pltpu.async_copy(dst_h.at[pl.ds(base, C1)], didx, sem)
            d1.wait()
            d2.wait()
            for g in range(C1 // L):
                sv = sidx[pl.ds(g * L, L)]
                dv = didx[pl.ds(g * L, L)]
                ri = iota + g * L
                for hh in range(4):
                    av = plsc.load_gather(atab, [sv * 4 + hh])
                    bv = plsc.load_gather(dtab, [dv * 4 + hh])
                    e = av + bv
                    e = jnp.maximum(e, NEG * e)
                    ex = jnp.exp(e - cmvec[col0 + hh])
                    plsc.store_scatter(
                        exbuf, [ri, jnp.full((L,), col0 + hh, i32)], ex)
            e1 = pltpu.async_copy(exbuf, ex_out.at[pl.ds(base, C1), :], sem)
            e2 = pltpu.async_copy(exbuf, den_sh.at[didx], sem2, add=True)
            e1.wait()
            e2.wait()
            return carry

        lax.fori_loop(0, NCH1, chunk_body, 0)

    plsc.subcore_barrier()
    pltpu.sync_copy(den_sh.at[pl.ds(s * RPT, RPT), :],
                    dp_h.at[c, pl.ds(s * RPT, RPT), :])


_sc_pass1 = functools.partial(
    pl.kernel,
    out_type=(
        jax.ShapeDtypeStruct((ET, HEADS), f32),
        jax.ShapeDtypeStruct((ET, HEADS), f32),
        jax.ShapeDtypeStruct((NC, NP, HEADS), f32),
    ),
    mesh=_MESH,
    compiler_params=pltpu.CompilerParams(
        needs_layout_passes=False, use_tc_tiling_on_sc=False),
    scratch_types=[
        pltpu.VMEM((N * 4,), f32),
        pltpu.VMEM((N * 4,), f32),
        pltpu.VMEM((C1,), i32),
        pltpu.VMEM((C1,), i32),
        pltpu.VMEM((C1, HEADS), f32),
        pltpu.VMEM((L,), f32),
        pltpu.SemaphoreType.DMA,
        pltpu.SemaphoreType.DMA,
        pltpu.VMEM_SHARED((NP, HEADS), f32),
    ],
)(_sc_pass1_body)


def _sc_pass2_body(src_h, dst_h, h_hbm, exa_h, exb_h,
                   op_h,
                   sidx, didx, didx2, exba, exbb, alphab, hrows, msgb,
                   semi1, semi2, semi3, semi4, semg, sem2, out_sh):
    c = lax.axis_index("c")
    s = lax.axis_index("s")
    wid = c * NS + s
    ebase = wid * EPW
    iota = lax.iota(i32, L)
    zero = jnp.zeros((L,), f32)

    for r in range(C2):
        for g in range(F // L):
            msgb[r, pl.ds(g * L, L)] = zero
    for r in range(0, RPT, C2):
        rows = min(C2, RPT - r)
        pltpu.sync_copy(msgb.at[pl.ds(0, rows), :],
                        out_sh.at[pl.ds(s * RPT + r, rows), :])
    plsc.subcore_barrier()

    def fire_inputs(base):
        pltpu.async_copy(src_h.at[pl.ds(base, C2)], sidx, semi1)
        pltpu.async_copy(dst_h.at[pl.ds(base, C2)], didx, semi2)
        pltpu.async_copy(exa_h.at[pl.ds(base, C2), :], exba, semi3)
        pltpu.async_copy(exb_h.at[pl.ds(base, C2), :], exbb, semi4)

    def drain_inputs():
        pltpu.make_async_copy(src_h.at[pl.ds(0, C2)], sidx, semi1).wait()
        pltpu.make_async_copy(dst_h.at[pl.ds(0, C2)], didx, semi2).wait()
        pltpu.make_async_copy(exa_h.at[pl.ds(0, C2), :], exba, semi3).wait()
        pltpu.make_async_copy(exb_h.at[pl.ds(0, C2), :], exbb, semi4).wait()

    fire_inputs(ebase)

    def chunk_body(ci, carry):
        pltpu.make_async_copy(src_h.at[pl.ds(0, C2)], sidx, semi1).wait()
        g1 = pltpu.async_copy(h_hbm.at[sidx], hrows, semg)
        pltpu.make_async_copy(dst_h.at[pl.ds(0, C2)], didx, semi2).wait()
        pltpu.make_async_copy(exa_h.at[pl.ds(0, C2), :], exba, semi3).wait()
        pltpu.make_async_copy(exb_h.at[pl.ds(0, C2), :], exbb, semi4).wait()
        for g in range(C2 // L):
            rv = iota + g * L
            for h in range(HEADS):
                hsp = jnp.full((L,), h, i32)
                exv = (plsc.load_gather(exba, [rv, hsp]) +
                       plsc.load_gather(exbb, [rv, hsp]))
                plsc.store_scatter(alphab, [rv, hsp], exv)
        for g in range(C2 // L):
            didx2[pl.ds(g * L, L)] = didx[pl.ds(g * L, L)]
        g1.wait()

        def edge_body(e, cc):
            arow = alphab[e, :]
            for g in range(HEADS):
                msgb[e, pl.ds(g * HID, HID)] = (
                    arow[g] * hrows[e, pl.ds(g * HID, HID)])
            return cc

        lax.fori_loop(0, C2, edge_body, 0)
        sc = pltpu.async_copy(msgb, out_sh.at[didx2], sem2, add=True)
        nci = jnp.minimum(ci + 1, NCH2 - 1)
        fire_inputs(ebase + nci * C2)
        sc.wait()
        return carry

    lax.fori_loop(0, NCH2, chunk_body, 0)
    drain_inputs()

    plsc.subcore_barrier()
    pltpu.sync_copy(out_sh.at[pl.ds(s * RPT, RPT), :],
                    op_h.at[c, pl.ds(s * RPT, RPT), :])


_sc_pass2 = functools.partial(
    pl.kernel,
    out_type=jax.ShapeDtypeStruct((NC, NP, F), f32),
    mesh=_MESH,
    compiler_params=pltpu.CompilerParams(
        needs_layout_passes=False, use_tc_tiling_on_sc=False),
    scratch_types=[
        pltpu.VMEM((C2,), i32),
        pltpu.VMEM((C2,), i32),
        pltpu.VMEM((C2,), i32),
        pltpu.VMEM((C2, HEADS), f32),
        pltpu.VMEM((C2, HEADS), f32),
        pltpu.VMEM((C2, L), f32),
        pltpu.VMEM((C2, F), f32),
        pltpu.VMEM((C2, F), f32),
        pltpu.SemaphoreType.DMA,
        pltpu.SemaphoreType.DMA,
        pltpu.SemaphoreType.DMA,
        pltpu.SemaphoreType.DMA,
        pltpu.SemaphoreType.DMA,
        pltpu.SemaphoreType.DMA,
        pltpu.VMEM_SHARED((NP, F), f32),
    ],
)(_sc_pass2_body)


def _make_aall(a_src, a_dst):
    jj = jnp.arange(F)
    blk = (jj[:, None] // HID == jnp.arange(HEADS)[None, :]).astype(f32)
    return jnp.concatenate(
        [blk * a_src.reshape(F)[:, None], blk * a_dst.reshape(F)[:, None]],
        axis=1)


def _sc_layer(h, asd, cm, src, dst):
    alo = asd[:, 0:4].reshape(-1)
    ahi = asd[:, 4:8].reshape(-1)
    dlo = asd[:, 8:12].reshape(-1)
    dhi = asd[:, 12:16].reshape(-1)
    cmv = cm.reshape(2 * HEADS)
    exa, exb, dp = _sc_pass1(src, dst, alo, ahi, dlo, dhi, cmv)
    return _sc_pass2(src, dst, h, exa, exb), dp


def kernel(x, edge_index, W0, as0, ad0, b0, g0, be0,
           W1, as1, ad1, b1, g1, be1, Wc, bc):
    pad = ET - E - N
    loops = jnp.arange(N, dtype=i32)
    src = jnp.concatenate(
        [edge_index[0], loops, jnp.zeros((pad,), i32)])
    dst = jnp.concatenate(
        [edge_index[1], loops, jnp.full((pad,), N, i32)])

    aall0 = _make_aall(as0, ad0)
    aall1 = _make_aall(as1, ad1)
    mavg = (jnp.arange(F)[:, None] % HID ==
            jnp.arange(HID)[None, :]).astype(f32) / HEADS

    b0r = b0.reshape(1, F)
    b1r = b1.reshape(1, HID)
    r8 = (jnp.arange(HEADS)[:, None] ==
          jnp.arange(F)[None, :] // HID).astype(f32)

    h0, asd0, cm0 = _tc_embed(x, W0, aall0)
    op0, dp0 = _sc_layer(h0, asd0, cm0, src, dst)
    s1, s2 = _tc_stat(op0, dp0, r8, b0r)
    h1, asd1, cm1 = _tc_main1(op0, dp0, r8, b0r, g0.reshape(1, F),
                              be0.reshape(1, F), s1, s2, W1, aall1)
    op1, dp1 = _sc_layer(h1, asd1, cm1, src, dst)

    s1f, s2f = _tc_statf(op1, dp1, r8, mavg, b1r)
    return _tc_final(op1, dp1, r8, mavg, b1r, g1.reshape(1, HID),
                     be1.reshape(1, HID), s1f, s2f, Wc, bc.reshape(1, 2))


# speedup vs baseline: 72.8127x; 1.1521x over previous
import functools

import jax
import jax.numpy as jnp
from jax import lax
from jax.experimental import pallas as pl
from jax.experimental.pallas import tpu as pltpu
from jax.experimental.pallas import tpu_sc as plsc

N = 10000
E = 320000
IN = 128
HID = 16
HEADS = 8
F = HEADS * HID
NEG = 0.2

NC = 2
NS = 16
L = 16
NW = NC * NS

EPW = 10368
ET = EPW * NW
NP = 10112
RPT = NP // NS
C1 = 128
C2 = 128
NCH1 = EPW // C1
NCH2 = EPW // C2

f32 = jnp.float32
i32 = jnp.int32


_RB = 400


def _embed_body(x_ref, w_ref, a_ref, h_ref, asd_ref, cm_ref, mx_ref):
    i = pl.program_id(0)
    h = jnp.dot(x_ref[...], w_ref[...], preferred_element_type=f32)
    h_ref[...] = h
    asd = jnp.dot(h, a_ref[...], preferred_element_type=f32)
    asd_ref[...] = asd
    bm = jnp.max(asd, axis=0, keepdims=True)

    @pl.when(i == 0)
    def _():
        mx_ref[...] = bm

    @pl.when(i > 0)
    def _():
        mx_ref[...] = jnp.maximum(mx_ref[...], bm)

    @pl.when(i == pl.num_programs(0) - 1)
    def _():
        m = mx_ref[...]
        c = m[:, :HEADS] + m[:, HEADS:]
        cl = jnp.maximum(c, NEG * c)
        cm_ref[...] = jnp.concatenate([cl, jnp.zeros_like(cl)], axis=1)


def _tc_embed(x, w, aall):
    return pl.pallas_call(
        _embed_body,
        grid=(N // _RB,),
        in_specs=[
            pl.BlockSpec((_RB, IN), lambda i: (i, 0)),
            pl.BlockSpec((IN, F), lambda i: (0, 0)),
            pl.BlockSpec((F, 2 * HEADS), lambda i: (0, 0)),
        ],
        out_specs=[
            pl.BlockSpec((_RB, F), lambda i: (i, 0)),
            pl.BlockSpec((_RB, 2 * HEADS), lambda i: (i, 0)),
            pl.BlockSpec((1, 2 * HEADS), lambda i: (0, 0)),
        ],
        out_shape=[
            jax.ShapeDtypeStruct((N, F), f32),
            jax.ShapeDtypeStruct((N, 2 * HEADS), f32),
            jax.ShapeDtypeStruct((1, 2 * HEADS), f32),
        ],
        scratch_shapes=[pltpu.VMEM((1, 2 * HEADS), f32)],
    )(x, w, aall)


def _stat_body(op_ref, dp_ref, r8_ref, b_ref, s1_ref, s2_ref):
    inv = 1.0 / (dp_ref[0, :N, :] + dp_ref[1, :N, :] + 1e-16)
    scale = jnp.dot(inv, r8_ref[...], preferred_element_type=f32)
    o = (op_ref[0, :N, :] + op_ref[1, :N, :]) * scale + b_ref[...]
    s1_ref[...] = jnp.sum(o, axis=0, keepdims=True)
    s2_ref[...] = jnp.sum(o * o, axis=0, keepdims=True)


def _tc_stat(op, dp, r8, b):
    d = b.shape[1]
    return pl.pallas_call(
        _stat_body,
        out_shape=[
            jax.ShapeDtypeStruct((1, d), f32),
            jax.ShapeDtypeStruct((1, d), f32),
        ],
    )(op, dp, r8, b)


def _main1_body(op_ref, dp_ref, r8_ref, b_ref, g_ref, be_ref, s1_ref, s2_ref,
                w_ref, a_ref, h_ref, asd_ref, cm_ref, mx_ref):
    i = pl.program_id(0)
    inv = 1.0 / (dp_ref[0] + dp_ref[1] + 1e-16)
    scale = jnp.dot(inv, r8_ref[...], preferred_element_type=f32)
    o = (op_ref[0] + op_ref[1]) * scale + b_ref[...]
    m = s1_ref[...] / N
    v = s2_ref[...] / N - m * m
    xn = g_ref[...] * (o - m) * lax.rsqrt(v + 1e-5) + be_ref[...]
    el = jnp.where(xn > 0, xn, jnp.exp(xn) - 1.0)
    h = jnp.dot(el, w_ref[...], preferred_element_type=f32)
    h_ref[...] = h
    asd = jnp.dot(h, a_ref[...], preferred_element_type=f32)
    asd_ref[...] = asd
    bm = jnp.max(asd, axis=0, keepdims=True)

    @pl.when(i == 0)
    def _():
        mx_ref[...] = bm

    @pl.when(i > 0)
    def _():
        mx_ref[...] = jnp.maximum(mx_ref[...], bm)

    @pl.when(i == pl.num_programs(0) - 1)
    def _():
        mm = mx_ref[...]
        c = mm[:, :HEADS] + mm[:, HEADS:]
        cl = jnp.maximum(c, NEG * c)
        cm_ref[...] = jnp.concatenate([cl, jnp.zeros_like(cl)], axis=1)


def _tc_main1(op, dp, r8, b, g, be, s1, s2, w, aall):
    return pl.pallas_call(
        _main1_body,
        grid=(N // _RB,),
        in_specs=[
            pl.BlockSpec((2, _RB, F), lambda i: (0, i, 0)),
            pl.BlockSpec((2, _RB, HEADS), lambda i: (0, i, 0)),
            pl.BlockSpec((HEADS, F), lambda i: (0, 0)),
            pl.BlockSpec((1, F), lambda i: (0, 0)),
            pl.BlockSpec((1, F), lambda i: (0, 0)),
            pl.BlockSpec((1, F), lambda i: (0, 0)),
            pl.BlockSpec((1, F), lambda i: (0, 0)),
            pl.BlockSpec((1, F), lambda i: (0, 0)),
            pl.BlockSpec((F, F), lambda i: (0, 0)),
            pl.BlockSpec((F, 2 * HEADS), lambda i: (0, 0)),
        ],
        out_specs=[
            pl.BlockSpec((_RB, F), lambda i: (i, 0)),
            pl.BlockSpec((_RB, 2 * HEADS), lambda i: (i, 0)),
            pl.BlockSpec((1, 2 * HEADS), lambda i: (0, 0)),
        ],
        out_shape=[
            jax.ShapeDtypeStruct((N, F), f32),
            jax.ShapeDtypeStruct((N, 2 * HEADS), f32),
            jax.ShapeDtypeStruct((1, 2 * HEADS), f32),
        ],
        scratch_shapes=[pltpu.VMEM((1, 2 * HEADS), f32)],
    )(op, dp, r8, b, g, be, s1, s2, w, aall)


def _statf_body(op_ref, dp_ref, r8_ref, mavg_ref, b_ref, s1_ref, s2_ref):
    inv = 1.0 / (dp_ref[0, :N, :] + dp_ref[1, :N, :] + 1e-16)
    scale = jnp.dot(inv, r8_ref[...], preferred_element_type=f32)
    o = (op_ref[0, :N, :] + op_ref[1, :N, :]) * scale
    hm = jnp.dot(o, mavg_ref[...], preferred_element_type=f32) + b_ref[...]
    s1_ref[...] = jnp.sum(hm, axis=0, keepdims=True)
    s2_ref[...] = jnp.sum(hm * hm, axis=0, keepdims=True)


def _tc_statf(op, dp, r8, mavg, b):
    return pl.pallas_call(
        _statf_body,
        out_shape=[
            jax.ShapeDtypeStruct((1, HID), f32),
            jax.ShapeDtypeStruct((1, HID), f32),
        ],
    )(op, dp, r8, mavg, b)


def _final_body(op_ref, dp_ref, r8_ref, mavg_ref, b_ref, g_ref, be_ref,
                s1_ref, s2_ref, wc_ref, bc_ref, out_ref):
    inv = 1.0 / (dp_ref[0, :N, :] + dp_ref[1, :N, :] + 1e-16)
    scale = jnp.dot(inv, r8_ref[...], preferred_element_type=f32)
    o = (op_ref[0, :N, :] + op_ref[1, :N, :]) * scale
    hm = jnp.dot(o, mavg_ref[...], preferred_element_type=f32) + b_ref[...]
    m = s1_ref[...] / N
    v = s2_ref[...] / N - m * m
    xn = g_ref[...] * (hm - m) * lax.rsqrt(v + 1e-5) + be_ref[...]
    out_ref[...] = jnp.dot(xn, wc_ref[...], preferred_element_type=f32) + bc_ref[...]


def _tc_final(op, dp, r8, mavg, b, g, be, s1, s2, wc, bc):
    return pl.pallas_call(
        _final_body,
        out_shape=jax.ShapeDtypeStruct((N, 2), f32),
    )(op, dp, r8, mavg, b, g, be, s1, s2, wc, bc)


_MESH = plsc.VectorSubcoreMesh(
    core_axis_name="c", subcore_axis_name="s", num_cores=NC, num_subcores=NS)


def _sc_pass1_body(src_h, dst_h, asrc_h, adst_h, cm_h,
                   ex_h, dp_h,
                   sidx, didx, didx2, arows, drows, exbuf, cm_v,
                   semi1, semi2, semga, semgd, seme1, seme2,
                   asrc_sh, adst_sh, den_sh):
    c = lax.axis_index("c")
    s = lax.axis_index("s")
    wid = c * NS + s
    ebase = wid * EPW
    iota = lax.iota(i32, L)
    zero = jnp.zeros((L,), f32)

    pltpu.sync_copy(cm_h, cm_v)
    cmvec = cm_v[...]

    @pl.when(s == 0)
    def _():
        pltpu.sync_copy(asrc_h, asrc_sh)

    @pl.when(s == 1)
    def _():
        pltpu.sync_copy(adst_h, adst_sh)

    for k in range(C1 * HEADS // L):
        flat = iota + k * L
        plsc.store_scatter(exbuf, [flat // HEADS, flat % HEADS], zero)
    for r in range(0, RPT, C1):
        rows = min(C1, RPT - r)
        pltpu.sync_copy(exbuf.at[pl.ds(0, rows), :],
                        den_sh.at[pl.ds(s * RPT + r, rows), :])
    plsc.subcore_barrier()

    def fire_inputs(base):
        pltpu.async_copy(src_h.at[pl.ds(base, C1)], sidx, semi1)
        pltpu.async_copy(dst_h.at[pl.ds(base, C1)], didx, semi2)

    fire_inputs(ebase)

    def chunk_body(ci, carry):
        pltpu.make_async_copy(src_h.at[pl.ds(0, C1)], sidx, semi1).wait()
        ga = pltpu.async_copy(asrc_sh.at[sidx], arows, semga)
        pltpu.make_async_copy(dst_h.at[pl.ds(0, C1)], didx, semi2).wait()
        gd = pltpu.async_copy(adst_sh.at[didx], drows, semgd)
        ga.wait()
        gd.wait()
        for g in range(C1 // L):
            rv = iota + g * L
            for h in range(HEADS):
                hsp = jnp.full((L,), h, i32)
                e = (plsc.load_gather(arows, [rv, hsp]) +
                     plsc.load_gather(drows, [rv, hsp]))
                e = jnp.maximum(e, NEG * e)
                ex = jnp.exp(e - cmvec[h])
                plsc.store_scatter(exbuf, [rv, hsp], ex)
        for g in range(C1 // L):
            didx2[pl.ds(g * L, L)] = didx[pl.ds(g * L, L)]
        base = ebase + ci * C1
        e1 = pltpu.async_copy(exbuf, ex_h.at[pl.ds(base, C1), :], seme1)
        e2 = pltpu.async_copy(exbuf, den_sh.at[didx2], seme2, add=True)
        nci = jnp.minimum(ci + 1, NCH1 - 1)
        fire_inputs(ebase + nci * C1)
        e1.wait()
        e2.wait()
        return carry

    lax.fori_loop(0, NCH1, chunk_body, 0)
    pltpu.make_async_copy(src_h.at[pl.ds(0, C1)], sidx, semi1).wait()
    pltpu.make_async_copy(dst_h.at[pl.ds(0, C1)], didx, semi2).wait()

    plsc.subcore_barrier()
    pltpu.sync_copy(den_sh.at[pl.ds(s * RPT, RPT), :],
                    dp_h.at[c, pl.ds(s * RPT, RPT), :])


_sc_pass1 = functools.partial(
    pl.kernel,
    out_type=(
        jax.ShapeDtypeStruct((ET, HEADS), f32),
        jax.ShapeDtypeStruct((NC, NP, HEADS), f32),
    ),
    mesh=_MESH,
    compiler_params=pltpu.CompilerParams(
        needs_layout_passes=False, use_tc_tiling_on_sc=False),
    scratch_types=[
        pltpu.VMEM((C1,), i32),
        pltpu.VMEM((C1,), i32),
        pltpu.VMEM((C1,), i32),
        pltpu.VMEM((C1, HEADS), f32),
        pltpu.VMEM((C1, HEADS), f32),
        pltpu.VMEM((C1, HEADS), f32),
        pltpu.VMEM((L,), f32),
        pltpu.SemaphoreType.DMA,
        pltpu.SemaphoreType.DMA,
        pltpu.SemaphoreType.DMA,
        pltpu.SemaphoreType.DMA,
        pltpu.SemaphoreType.DMA,
        pltpu.SemaphoreType.DMA,
        pltpu.VMEM_SHARED((N, HEADS), f32),
        pltpu.VMEM_SHARED((N, HEADS), f32),
        pltpu.VMEM_SHARED((NP, HEADS), f32),
    ],
)(_sc_pass1_body)


def _sc_pass2_body(src_h, dst_h, h_hbm, exa_h,
                   op_h,
                   sidx, didx, didx2, exba, alphab, hrows, msgb,
                   semi1, semi2, semi3, semg, sem2, out_sh):
    c = lax.axis_index("c")
    s = lax.axis_index("s")
    wid = c * NS + s
    ebase = wid * EPW
    iota = lax.iota(i32, L)
    zero = jnp.zeros((L,), f32)

    for r in range(C2):
        for g in range(F // L):
            msgb[r, pl.ds(g * L, L)] = zero
    for r in range(0, RPT, C2):
        rows = min(C2, RPT - r)
        pltpu.sync_copy(msgb.at[pl.ds(0, rows), :],
                        out_sh.at[pl.ds(s * RPT + r, rows), :])
    plsc.subcore_barrier()

    def fire_inputs(base):
        pltpu.async_copy(src_h.at[pl.ds(base, C2)], sidx, semi1)
        pltpu.async_copy(dst_h.at[pl.ds(base, C2)], didx, semi2)
        pltpu.async_copy(exa_h.at[pl.ds(base, C2), :], exba, semi3)

    def drain_inputs():
        pltpu.make_async_copy(src_h.at[pl.ds(0, C2)], sidx, semi1).wait()
        pltpu.make_async_copy(dst_h.at[pl.ds(0, C2)], didx, semi2).wait()
        pltpu.make_async_copy(exa_h.at[pl.ds(0, C2), :], exba, semi3).wait()

    fire_inputs(ebase)

    def chunk_body(ci, carry):
        pltpu.make_async_copy(src_h.at[pl.ds(0, C2)], sidx, semi1).wait()
        g1 = pltpu.async_copy(h_hbm.at[sidx], hrows, semg)
        pltpu.make_async_copy(dst_h.at[pl.ds(0, C2)], didx, semi2).wait()
        pltpu.make_async_copy(exa_h.at[pl.ds(0, C2), :], exba, semi3).wait()
        for g in range(C2 // L):
            rv = iota + g * L
            for h in range(HEADS):
                hsp = jnp.full((L,), h, i32)
                plsc.store_scatter(alphab, [rv, hsp],
                                   plsc.load_gather(exba, [rv, hsp]))
        for g in range(C2 // L):
            didx2[pl.ds(g * L, L)] = didx[pl.ds(g * L, L)]
        g1.wait()

        def edge_body(e, cc):
            arow = alphab[e, :]
            for g in range(HEADS):
                msgb[e, pl.ds(g * HID, HID)] = (
                    arow[g] * hrows[e, pl.ds(g * HID, HID)])
            return cc

        lax.fori_loop(0, C2, edge_body, 0)
        sc = pltpu.async_copy(msgb, out_sh.at[didx2], sem2, add=True)
        nci = jnp.minimum(ci + 1, NCH2 - 1)
        fire_inputs(ebase + nci * C2)
        sc.wait()
        return carry

    lax.fori_loop(0, NCH2, chunk_body, 0)
    drain_inputs()

    plsc.subcore_barrier()
    pltpu.sync_copy(out_sh.at[pl.ds(s * RPT, RPT), :],
                    op_h.at[c, pl.ds(s * RPT, RPT), :])


_sc_pass2 = functools.partial(
    pl.kernel,
    out_type=jax.ShapeDtypeStruct((NC, NP, F), f32),
    mesh=_MESH,
    compiler_params=pltpu.CompilerParams(
        needs_layout_passes=False, use_tc_tiling_on_sc=False),
    scratch_types=[
        pltpu.VMEM((C2,), i32),
        pltpu.VMEM((C2,), i32),
        pltpu.VMEM((C2,), i32),
        pltpu.VMEM((C2, HEADS), f32),
        pltpu.VMEM((C2, L), f32),
        pltpu.VMEM((C2, F), f32),
        pltpu.VMEM((C2, F), f32),
        pltpu.SemaphoreType.DMA,
        pltpu.SemaphoreType.DMA,
        pltpu.SemaphoreType.DMA,
        pltpu.SemaphoreType.DMA,
        pltpu.SemaphoreType.DMA,
        pltpu.VMEM_SHARED((NP, F), f32),
    ],
)(_sc_pass2_body)


def _make_aall(a_src, a_dst):
    jj = jnp.arange(F)
    blk = (jj[:, None] // HID == jnp.arange(HEADS)[None, :]).astype(f32)
    return jnp.concatenate(
        [blk * a_src.reshape(F)[:, None], blk * a_dst.reshape(F)[:, None]],
        axis=1)


def _sc_layer(h, asd, cm, src, dst):
    asrc = asd[:, :HEADS]
    adst = asd[:, HEADS:]
    cmv = cm.reshape(2 * HEADS)
    exa, dp = _sc_pass1(src, dst, asrc, adst, cmv)
    return _sc_pass2(src, dst, h, exa), dp


def kernel(x, edge_index, W0, as0, ad0, b0, g0, be0,
           W1, as1, ad1, b1, g1, be1, Wc, bc):
    pad = ET - E - N
    loops = jnp.arange(N, dtype=i32)
    src = jnp.concatenate(
        [edge_index[0], loops, jnp.zeros((pad,), i32)])
    dst = jnp.concatenate(
        [edge_index[1], loops, jnp.full((pad,), N, i32)])

    aall0 = _make_aall(as0, ad0)
    aall1 = _make_aall(as1, ad1)
    mavg = (jnp.arange(F)[:, None] % HID ==
            jnp.arange(HID)[None, :]).astype(f32) / HEADS

    b0r = b0.reshape(1, F)
    b1r = b1.reshape(1, HID)
    r8 = (jnp.arange(HEADS)[:, None] ==
          jnp.arange(F)[None, :] // HID).astype(f32)

    h0, asd0, cm0 = _tc_embed(x, W0, aall0)
    op0, dp0 = _sc_layer(h0, asd0, cm0, src, dst)
    s1, s2 = _tc_stat(op0, dp0, r8, b0r)
    h1, asd1, cm1 = _tc_main1(op0, dp0, r8, b0r, g0.reshape(1, F),
                              be0.reshape(1, F), s1, s2, W1, aall1)
    op1, dp1 = _sc_layer(h1, asd1, cm1, src, dst)

    s1f, s2f = _tc_statf(op1, dp1, r8, mavg, b1r)
    return _tc_final(op1, dp1, r8, mavg, b1r, g1.reshape(1, HID),
                     be1.reshape(1, HID), s1f, s2f, Wc, bc.reshape(1, 2))
